# baseline (speedup 1.0000x reference)
import functools

import jax
import jax.numpy as jnp
from jax import lax
from jax.experimental import pallas as pl
from jax.experimental.pallas import tpu as pltpu
from jax.experimental.pallas import tpu_sc as plsc

_D = 64
_SCALE = 8.0
_NC, _NS = 2, 16
_NW = _NC * _NS
_B = 819200
_CHUNK = 512
_NBUF = 2
_SB = 25
_NBB = 32
_K = 5
_SBP = _SB // _K
_BP = _B // _K


def _sc_gather(xs, table, part):
    b_per_w = _BP // _NW
    nchunks = b_per_w // _CHUNK

    mesh = plsc.VectorSubcoreMesh(core_axis_name="c", subcore_axis_name="s")

    @functools.partial(
        pl.kernel,
        out_type=jax.ShapeDtypeStruct((_BP, _D), jnp.float32),
        mesh=mesh,
        scratch_types=[
            pltpu.VMEM((b_per_w,), jnp.int32),
            pltpu.VMEM((b_per_w,), jnp.int32),
            pltpu.VMEM((_NBUF, _CHUNK, _D), jnp.float32),
            pltpu.SemaphoreType.DMA,
            pltpu.SemaphoreType.DMA,
            pltpu.SemaphoreType.DMA,
            pltpu.SemaphoreType.DMA,
        ],
        compiler_params=pltpu.CompilerParams(
            use_tc_tiling_on_sc=False, needs_layout_passes=False
        ),
    )
    def emb(idx_hbm, table_hbm, out_hbm, idx_r, idx_v, rows_v, g0, g1, s0, s1):
        wid = lax.axis_index("s") * _NC + lax.axis_index("c")
        base = wid * b_per_w
        pltpu.sync_copy(
            idx_hbm.at[pl.ds(part * _BP + base, b_per_w)], idx_r
        )

        ii2 = lax.iota(jnp.int32, 16) * 2

        @plsc.parallel_loop(0, b_per_w // 256)
        def _(blk):
            for g in range(8):
                src = blk * 256 + g * 16
                a = idx_r[pl.ds(src, 16)]
                b = idx_r[pl.ds(src + 128, 16)]
                pos = ii2 + (blk * 256 + g * 32)
                plsc.store_scatter(idx_v, [pos], a)
                plsc.store_scatter(idx_v, [pos + 1], b)

        gsem = (g0, g1)
        ssem = (s0, s1)

        def start_gather(slot, c):
            pltpu.make_async_copy(
                table_hbm.at[idx_v.at[pl.ds(c * _CHUNK, _CHUNK)]],
                rows_v.at[slot],
                gsem[slot],
            ).start()

        def wait_gather(slot):
            pltpu.make_async_copy(
                table_hbm.at[idx_v.at[pl.ds(0, _CHUNK)]],
                rows_v.at[slot],
                gsem[slot],
            ).wait()

        def start_scatter(slot, c):
            pltpu.make_async_copy(
                rows_v.at[slot],
                out_hbm.at[pl.ds(base + c * _CHUNK, _CHUNK)],
                ssem[slot],
            ).start()

        def wait_scatter(slot):
            pltpu.make_async_copy(
                rows_v.at[slot],
                out_hbm.at[pl.ds(base, _CHUNK)],
                ssem[slot],
            ).wait()

        start_gather(0, 0)
        start_gather(1, 1)

        @pl.loop(0, (nchunks - 2) // 2)
        def _(i):
            c0 = i * 2
            for b in range(_NBUF):
                wait_gather(b)
                start_scatter(b, c0 + b)
            for b in range(_NBUF):
                wait_scatter(b)
                start_gather(b, c0 + 2 + b)

        for b in range(_NBUF):
            wait_gather(b)
            start_scatter(b, nchunks - 2 + b)
        for b in range(_NBUF):
            wait_scatter(b)

    return emb(xs, table)


def _tc_unpack(rows_ref, out_ref):
    for a in range(4):
        pt = jnp.transpose(rows_ref[a], (1, 0)) * _SCALE
        for j in range(2):
            out_ref[2 * a + j, :, 0, :, :] = pt[
                j * _D : (j + 1) * _D, :
            ].reshape(8, 8, 128)


def _tc_body_first(rows_ref, out_ref):
    _tc_unpack(rows_ref, out_ref)


def _tc_body(prev_ref, rows_ref, out_ref):
    del prev_ref
    _tc_unpack(rows_ref, out_ref)


def _tc_transpose(rows3, part, prev):
    out_shape = jax.ShapeDtypeStruct((200, 8, _NBB, 8, 128), jnp.float32)
    grid = (_SBP, _NBB)

    def in_map(sb, bb):
        return (sb * _NBB + bb, 0, 0)

    def out_map2(sb, bb, _part=part):
        return (_part * _SBP + sb, 0, bb, 0, 0)

    rows_spec = pl.BlockSpec((4, 128, 128), in_map)
    out_spec = pl.BlockSpec((8, 8, 1, 8, 128), out_map2)
    if prev is None:
        return pl.pallas_call(
            _tc_body_first,
            grid=grid,
            in_specs=[rows_spec],
            out_specs=out_spec,
            out_shape=out_shape,
            compiler_params=pltpu.CompilerParams(
                dimension_semantics=("arbitrary", "arbitrary")
            ),
        )(rows3)
    return pl.pallas_call(
        _tc_body,
        grid=grid,
        in_specs=[pl.BlockSpec(memory_space=pl.ANY), rows_spec],
        out_specs=out_spec,
        out_shape=out_shape,
        input_output_aliases={0: 0},
        compiler_params=pltpu.CompilerParams(
            dimension_semantics=("arbitrary", "arbitrary")
        ),
    )(prev, rows3)


@jax.jit
def _lookup(xs, table):
    rows = [_sc_gather(xs, table, p) for p in range(_K)]
    out = None
    for p in range(_K):
        out = _tc_transpose(rows[p].reshape(_BP // 256, 128, 128), p, out)
    return out


def kernel(x, table):
    xs = x.astype(jnp.int32).reshape(_NBB, 128, _SB, 8).transpose(2, 0, 3, 1)
    out5 = _lookup(xs.reshape(-1), table)
    return out5.transpose(2, 4, 0, 1, 3).reshape(4096, 200, _D)


# speedup vs baseline: 1.1912x; 1.1912x over previous
import functools

import jax
import jax.numpy as jnp
from jax import lax
from jax.experimental import pallas as pl
from jax.experimental.pallas import tpu as pltpu
from jax.experimental.pallas import tpu_sc as plsc

_D = 64
_SCALE = 8.0
_NC, _NS = 2, 16
_NW = _NC * _NS
_B = 819200
_CHUNK = 512
_NBUF = 2
_SB = 25
_NBB = 32
_K = 5
_SBP = _SB // _K
_BP = _B // _K


def _sc_gather(xs, table, part):
    b_per_w = _BP // _NW
    nchunks = b_per_w // _CHUNK

    mesh = plsc.VectorSubcoreMesh(core_axis_name="c", subcore_axis_name="s")

    @functools.partial(
        pl.kernel,
        out_type=jax.ShapeDtypeStruct((_BP, _D), jnp.float32),
        mesh=mesh,
        scratch_types=[
            pltpu.VMEM((b_per_w,), jnp.int32),
            pltpu.VMEM((b_per_w,), jnp.int32),
            pltpu.VMEM((_NBUF, _CHUNK, _D), jnp.float32),
            pltpu.SemaphoreType.DMA,
            pltpu.SemaphoreType.DMA,
            pltpu.SemaphoreType.DMA,
            pltpu.SemaphoreType.DMA,
        ],
        compiler_params=pltpu.CompilerParams(
            use_tc_tiling_on_sc=False, needs_layout_passes=False
        ),
    )
    def emb(idx_hbm, table_hbm, out_hbm, idx_r, idx_v, rows_v, g0, g1, s0, s1):
        wid = lax.axis_index("s") * _NC + lax.axis_index("c")
        base = wid * b_per_w
        pltpu.sync_copy(
            idx_hbm.at[pl.ds(part * _BP + base, b_per_w)], idx_r
        )

        ii2 = lax.iota(jnp.int32, 16) * 2

        @plsc.parallel_loop(0, b_per_w // 256)
        def _(blk):
            for g in range(8):
                src = blk * 256 + g * 16
                a = idx_r[pl.ds(src, 16)]
                b = idx_r[pl.ds(src + 128, 16)]
                pos = ii2 + (blk * 256 + g * 32)
                plsc.store_scatter(idx_v, [pos], a)
                plsc.store_scatter(idx_v, [pos + 1], b)

        gsem = (g0, g1)
        ssem = (s0, s1)

        def start_gather(slot, c):
            pltpu.make_async_copy(
                table_hbm.at[idx_v.at[pl.ds(c * _CHUNK, _CHUNK)]],
                rows_v.at[slot],
                gsem[slot],
            ).start()

        def wait_gather(slot):
            pltpu.make_async_copy(
                table_hbm.at[idx_v.at[pl.ds(0, _CHUNK)]],
                rows_v.at[slot],
                gsem[slot],
            ).wait()

        def start_scatter(slot, c):
            pltpu.make_async_copy(
                rows_v.at[slot],
                out_hbm.at[pl.ds(base + c * _CHUNK, _CHUNK)],
                ssem[slot],
            ).start()

        def wait_scatter(slot):
            pltpu.make_async_copy(
                rows_v.at[slot],
                out_hbm.at[pl.ds(base, _CHUNK)],
                ssem[slot],
            ).wait()

        start_gather(0, 0)
        start_gather(1, 1)

        @pl.loop(0, (nchunks - 2) // 2)
        def _(i):
            c0 = i * 2
            for b in range(_NBUF):
                wait_gather(b)
                start_scatter(b, c0 + b)
            for b in range(_NBUF):
                wait_scatter(b)
                start_gather(b, c0 + 2 + b)

        for b in range(_NBUF):
            wait_gather(b)
            start_scatter(b, nchunks - 2 + b)
        for b in range(_NBUF):
            wait_scatter(b)

    return emb(xs, table)


def _tc_unpack(rows_ref, out_ref):
    for tt in range(8):
        pt = jnp.transpose(rows_ref[tt], (1, 0)) * _SCALE
        for j in range(2):
            out_ref[2 * (tt % 4) + j, :, tt // 4, :, :] = pt[
                j * _D : (j + 1) * _D, :
            ].reshape(8, 8, 128)


def _tc_body_first(rows_ref, out_ref):
    _tc_unpack(rows_ref, out_ref)


def _tc_body(prev_ref, rows_ref, out_ref):
    del prev_ref
    _tc_unpack(rows_ref, out_ref)


def _tc_transpose(rows3, part, prev):
    out_shape = jax.ShapeDtypeStruct((200, 8, _NBB, 8, 128), jnp.float32)
    grid = (_SBP, _NBB // 2)

    def in_map(sb, bc):
        return (sb * (_NBB // 2) + bc, 0, 0)

    def out_map2(sb, bc, _part=part):
        return (_part * _SBP + sb, 0, bc, 0, 0)

    rows_spec = pl.BlockSpec((8, 128, 128), in_map)
    out_spec = pl.BlockSpec((8, 8, 2, 8, 128), out_map2)
    if prev is None:
        return pl.pallas_call(
            _tc_body_first,
            grid=grid,
            in_specs=[rows_spec],
            out_specs=out_spec,
            out_shape=out_shape,
            compiler_params=pltpu.CompilerParams(
                dimension_semantics=("arbitrary", "arbitrary")
            ),
        )(rows3)
    return pl.pallas_call(
        _tc_body,
        grid=grid,
        in_specs=[pl.BlockSpec(memory_space=pl.ANY), rows_spec],
        out_specs=out_spec,
        out_shape=out_shape,
        input_output_aliases={0: 0},
        compiler_params=pltpu.CompilerParams(
            dimension_semantics=("arbitrary", "arbitrary")
        ),
    )(prev, rows3)


@jax.jit
def _lookup(xs, table):
    t2 = jax.lax.optimization_barrier(table.reshape(500000, 2 * _D))
    table = t2.reshape(1000000, _D)
    rows = [_sc_gather(xs, table, p) for p in range(_K)]
    out = None
    for p in range(_K):
        out = _tc_transpose(rows[p].reshape(_BP // 256, 128, 128), p, out)
    return out


def kernel(x, table):
    xs = x.astype(jnp.int32).reshape(_NBB, 128, _SB, 8).transpose(2, 0, 3, 1)
    out5 = _lookup(xs.reshape(-1), table)
    return out5.transpose(2, 4, 0, 1, 3).reshape(4096, 200, _D)


# speedup vs baseline: 1.3994x; 1.1748x over previous
import functools

import jax
import jax.numpy as jnp
from jax import lax
from jax.experimental import pallas as pl
from jax.experimental.pallas import tpu as pltpu
from jax.experimental.pallas import tpu_sc as plsc

_D = 64
_SCALE = 8.0
_NC, _NS = 2, 16
_NW = _NC * _NS
_B = 819200
_CHUNK = 512
_NBUF = 2
_SB = 25
_NBB = 32
_K = 5
_SBP = _SB // _K
_BP = _B // _K


def _sc_gather(xs, table, part):
    b_per_w = _BP // _NW
    nchunks = b_per_w // _CHUNK

    mesh = plsc.VectorSubcoreMesh(core_axis_name="c", subcore_axis_name="s")

    @functools.partial(
        pl.kernel,
        out_type=jax.ShapeDtypeStruct((_BP, _D), jnp.float32),
        mesh=mesh,
        scratch_types=[
            pltpu.VMEM((b_per_w,), jnp.int32),
            pltpu.VMEM((b_per_w,), jnp.int32),
            pltpu.VMEM((_NBUF, _CHUNK, _D), jnp.float32),
            pltpu.SemaphoreType.DMA,
            pltpu.SemaphoreType.DMA,
            pltpu.SemaphoreType.DMA,
            pltpu.SemaphoreType.DMA,
        ],
        compiler_params=pltpu.CompilerParams(
            use_tc_tiling_on_sc=False, needs_layout_passes=False
        ),
    )
    def emb(idx_hbm, table_hbm, out_hbm, idx_r, idx_v, rows_v, g0, g1, s0, s1):
        wid = lax.axis_index("s") * _NC + lax.axis_index("c")
        base = wid * b_per_w
        pltpu.sync_copy(
            idx_hbm.at[pl.ds(part * _BP + base, b_per_w)], idx_r
        )

        def remap(v):
            return (v & -2048) + ((v & 1023) << 1) + ((v >> 10) & 1)

        ii2 = lax.iota(jnp.int32, 16) * 2

        @plsc.parallel_loop(0, b_per_w // 256)
        def _(blk):
            for g in range(8):
                src = blk * 256 + g * 16
                a = remap(idx_r[pl.ds(src, 16)])
                b = remap(idx_r[pl.ds(src + 128, 16)])
                pos = ii2 + (blk * 256 + g * 32)
                plsc.store_scatter(idx_v, [pos], a)
                plsc.store_scatter(idx_v, [pos + 1], b)

        gsem = (g0, g1)
        ssem = (s0, s1)

        def start_gather(slot, c):
            pltpu.make_async_copy(
                table_hbm.at[idx_v.at[pl.ds(c * _CHUNK, _CHUNK)]],
                rows_v.at[slot],
                gsem[slot],
            ).start()

        def wait_gather(slot):
            pltpu.make_async_copy(
                table_hbm.at[idx_v.at[pl.ds(0, _CHUNK)]],
                rows_v.at[slot],
                gsem[slot],
            ).wait()

        def start_scatter(slot, c):
            pltpu.make_async_copy(
                rows_v.at[slot],
                out_hbm.at[pl.ds(base + c * _CHUNK, _CHUNK)],
                ssem[slot],
            ).start()

        def wait_scatter(slot):
            pltpu.make_async_copy(
                rows_v.at[slot],
                out_hbm.at[pl.ds(base, _CHUNK)],
                ssem[slot],
            ).wait()

        start_gather(0, 0)
        start_gather(1, 1)

        @pl.loop(0, (nchunks - 2) // 2)
        def _(i):
            c0 = i * 2
            for b in range(_NBUF):
                wait_gather(b)
                start_scatter(b, c0 + b)
            for b in range(_NBUF):
                wait_scatter(b)
                start_gather(b, c0 + 2 + b)

        for b in range(_NBUF):
            wait_gather(b)
            start_scatter(b, nchunks - 2 + b)
        for b in range(_NBUF):
            wait_scatter(b)

    return emb(xs, table)


def _tc_relayout_body(tref, oref):
    blk = tref[...] * _SCALE
    lo = jnp.transpose(blk[:, :1024], (1, 0))
    hi = jnp.transpose(blk[:, 1024:], (1, 0))
    oref[...] = jnp.concatenate([lo, hi], axis=1).reshape(128, 8, 128)


def _tc_relayout(table_t):
    return pl.pallas_call(
        _tc_relayout_body,
        grid=(489,),
        in_specs=[pl.BlockSpec((_D, 2048), lambda j: (0, j))],
        out_specs=pl.BlockSpec((128, 8, 128), lambda j: (j, 0, 0)),
        out_shape=jax.ShapeDtypeStruct((62592, 8, 128), jnp.float32),
        compiler_params=pltpu.CompilerParams(
            dimension_semantics=("arbitrary",)
        ),
    )(table_t)


def _tc_unpack(rows_ref, out_ref):
    for tt in range(8):
        pt = jnp.transpose(rows_ref[tt], (1, 0))
        for j in range(2):
            out_ref[2 * (tt % 4) + j, :, tt // 4, :, :] = pt[
                j * _D : (j + 1) * _D, :
            ].reshape(8, 8, 128)


def _tc_body_first(rows_ref, out_ref):
    _tc_unpack(rows_ref, out_ref)


def _tc_body(prev_ref, rows_ref, out_ref):
    del prev_ref
    _tc_unpack(rows_ref, out_ref)


def _tc_transpose(rows3, part, prev):
    out_shape = jax.ShapeDtypeStruct((200, 8, _NBB, 8, 128), jnp.float32)
    grid = (_SBP, _NBB // 2)

    def in_map(sb, bc):
        return (sb * (_NBB // 2) + bc, 0, 0)

    def out_map2(sb, bc, _part=part):
        return (_part * _SBP + sb, 0, bc, 0, 0)

    rows_spec = pl.BlockSpec((8, 128, 128), in_map)
    out_spec = pl.BlockSpec((8, 8, 2, 8, 128), out_map2)
    if prev is None:
        return pl.pallas_call(
            _tc_body_first,
            grid=grid,
            in_specs=[rows_spec],
            out_specs=out_spec,
            out_shape=out_shape,
            compiler_params=pltpu.CompilerParams(
                dimension_semantics=("arbitrary", "arbitrary")
            ),
        )(rows3)
    return pl.pallas_call(
        _tc_body,
        grid=grid,
        in_specs=[pl.BlockSpec(memory_space=pl.ANY), rows_spec],
        out_specs=out_spec,
        out_shape=out_shape,
        input_output_aliases={0: 0},
        compiler_params=pltpu.CompilerParams(
            dimension_semantics=("arbitrary", "arbitrary")
        ),
    )(prev, rows3)


@jax.jit
def _lookup(xs, table):
    table = _tc_relayout(table.T).reshape(1001472, _D)
    rows = [_sc_gather(xs, table, p) for p in range(_K)]
    out = None
    for p in range(_K):
        out = _tc_transpose(rows[p].reshape(_BP // 256, 128, 128), p, out)
    return out


def kernel(x, table):
    xs = x.astype(jnp.int32).reshape(_NBB, 128, _SB, 8).transpose(2, 0, 3, 1)
    out5 = _lookup(xs.reshape(-1), table)
    return out5.transpose(2, 4, 0, 1, 3).reshape(4096, 200, _D)


# speedup vs baseline: 1.6248x; 1.1611x over previous
import functools

import jax
import jax.numpy as jnp
from jax import lax
from jax.experimental import pallas as pl
from jax.experimental.pallas import tpu as pltpu
from jax.experimental.pallas import tpu_sc as plsc

_D = 64
_SCALE = 8.0
_NC, _NS = 2, 16
_NW = _NC * _NS
_B = 819200
_CHUNK = 512
_NBUF = 2
_SB = 25
_NBB = 32
_K = 5
_SBP = _SB // _K
_BP = _B // _K


def _sc_gather(xs, table, part):
    b_per_w = _BP // _NW
    nchunks = b_per_w // _CHUNK

    mesh = plsc.VectorSubcoreMesh(core_axis_name="c", subcore_axis_name="s")

    @functools.partial(
        pl.kernel,
        out_type=jax.ShapeDtypeStruct((_BP, _D), jnp.float32),
        mesh=mesh,
        scratch_types=[
            pltpu.VMEM((b_per_w,), jnp.int32),
            pltpu.VMEM((b_per_w,), jnp.int32),
            pltpu.VMEM((_NBUF, _CHUNK, _D), jnp.float32),
            pltpu.SemaphoreType.DMA,
            pltpu.SemaphoreType.DMA,
            pltpu.SemaphoreType.DMA,
            pltpu.SemaphoreType.DMA,
        ],
        compiler_params=pltpu.CompilerParams(
            use_tc_tiling_on_sc=False, needs_layout_passes=False
        ),
    )
    def emb(idx_hbm, table_hbm, out_hbm, idx_r, idx_v, rows_v, g0, g1, s0, s1):
        wid = lax.axis_index("s") * _NC + lax.axis_index("c")
        base = wid * b_per_w
        pltpu.sync_copy(
            idx_hbm.at[pl.ds(part * _BP + base, b_per_w)], idx_r
        )

        def remap(v):
            return (v & -2048) + ((v & 1023) << 1) + ((v >> 10) & 1)

        ii2 = lax.iota(jnp.int32, 16) * 2

        @plsc.parallel_loop(0, b_per_w // 256)
        def _(blk):
            for g in range(8):
                src = blk * 256 + g * 16
                a = remap(idx_r[pl.ds(src, 16)])
                b = remap(idx_r[pl.ds(src + 128, 16)])
                pos = ii2 + (blk * 256 + g * 32)
                plsc.store_scatter(idx_v, [pos], a)
                plsc.store_scatter(idx_v, [pos + 1], b)

        gsem = (g0, g1)
        ssem = (s0, s1)

        def start_gather(slot, c):
            pltpu.make_async_copy(
                table_hbm.at[idx_v.at[pl.ds(c * _CHUNK, _CHUNK)]],
                rows_v.at[slot],
                gsem[slot],
            ).start()

        def wait_gather(slot):
            pltpu.make_async_copy(
                table_hbm.at[idx_v.at[pl.ds(0, _CHUNK)]],
                rows_v.at[slot],
                gsem[slot],
            ).wait()

        def start_scatter(slot, c):
            pltpu.make_async_copy(
                rows_v.at[slot],
                out_hbm.at[pl.ds(base + c * _CHUNK, _CHUNK)],
                ssem[slot],
            ).start()

        def wait_scatter(slot):
            pltpu.make_async_copy(
                rows_v.at[slot],
                out_hbm.at[pl.ds(base, _CHUNK)],
                ssem[slot],
            ).wait()

        start_gather(0, 0)
        start_gather(1, 1)

        @pl.loop(0, (nchunks - 2) // 2)
        def _(i):
            c0 = i * 2
            for b in range(_NBUF):
                wait_gather(b)
                start_scatter(b, c0 + b)
            for b in range(_NBUF):
                wait_scatter(b)
                start_gather(b, c0 + 2 + b)

        for b in range(_NBUF):
            wait_gather(b)
            start_scatter(b, nchunks - 2 + b)
        for b in range(_NBUF):
            wait_scatter(b)

    return emb(xs, table)


def _tc_relayout_body(tref, oref):
    blk = tref[...] * _SCALE
    for h in range(2):
        sub = blk[:, h * 2048 : (h + 1) * 2048]
        lo = jnp.transpose(sub[:, :1024], (1, 0))
        hi = jnp.transpose(sub[:, 1024:], (1, 0))
        oref[h * 128 : (h + 1) * 128] = jnp.concatenate(
            [lo, hi], axis=1
        ).reshape(128, 8, 128)


def _tc_relayout(table_t):
    return pl.pallas_call(
        _tc_relayout_body,
        grid=(245,),
        in_specs=[pl.BlockSpec((_D, 4096), lambda j: (0, j))],
        out_specs=pl.BlockSpec((256, 8, 128), lambda j: (j, 0, 0)),
        out_shape=jax.ShapeDtypeStruct((62720, 8, 128), jnp.float32),
        compiler_params=pltpu.CompilerParams(
            dimension_semantics=("arbitrary",)
        ),
    )(table_t)


def _tc_unpack(rows_ref, out_ref):
    for tt in range(8):
        pt = jnp.transpose(rows_ref[tt], (1, 0))
        for j in range(2):
            out_ref[2 * (tt % 4) + j, :, tt // 4, :, :] = pt[
                j * _D : (j + 1) * _D, :
            ].reshape(8, 8, 128)


def _tc_body_first(rows_ref, out_ref):
    _tc_unpack(rows_ref, out_ref)


def _tc_body(prev_ref, rows_ref, out_ref):
    del prev_ref
    _tc_unpack(rows_ref, out_ref)


def _tc_transpose(rows3, part, prev):
    out_shape = jax.ShapeDtypeStruct((200, 8, _NBB, 8, 128), jnp.float32)
    grid = (_SBP, _NBB // 2)

    def in_map(sb, bc):
        return (sb * (_NBB // 2) + bc, 0, 0)

    def out_map2(sb, bc, _part=part):
        return (_part * _SBP + sb, 0, bc, 0, 0)

    rows_spec = pl.BlockSpec((8, 128, 128), in_map)
    out_spec = pl.BlockSpec((8, 8, 2, 8, 128), out_map2)
    if prev is None:
        return pl.pallas_call(
            _tc_body_first,
            grid=grid,
            in_specs=[rows_spec],
            out_specs=out_spec,
            out_shape=out_shape,
            compiler_params=pltpu.CompilerParams(
                dimension_semantics=("arbitrary", "arbitrary")
            ),
        )(rows3)
    return pl.pallas_call(
        _tc_body,
        grid=grid,
        in_specs=[pl.BlockSpec(memory_space=pl.ANY), rows_spec],
        out_specs=out_spec,
        out_shape=out_shape,
        input_output_aliases={0: 0},
        compiler_params=pltpu.CompilerParams(
            dimension_semantics=("arbitrary", "arbitrary")
        ),
    )(prev, rows3)


@jax.jit
def _lookup(xs, table):
    table = _tc_relayout(table.T).reshape(1003520, _D)
    rows = [_sc_gather(xs, table, p) for p in range(_K)]
    out = None
    for p in range(_K):
        out = _tc_transpose(rows[p].reshape(_BP // 256, 128, 128), p, out)
    return out


def kernel(x, table):
    xs = x.astype(jnp.int32).reshape(_NBB, 128, _SB, 8).transpose(2, 0, 3, 1)
    out5 = _lookup(xs.reshape(-1), table)
    return out5.transpose(2, 4, 0, 1, 3).reshape(4096, 200, _D)


# speedup vs baseline: 1.7831x; 1.0975x over previous
import functools

import jax
import jax.numpy as jnp
from jax import lax
from jax.experimental import pallas as pl
from jax.experimental.pallas import tpu as pltpu
from jax.experimental.pallas import tpu_sc as plsc

_D = 64
_SCALE = 8.0
_NC, _NS = 2, 16
_NW = _NC * _NS
_B = 819200
_CHUNK = 512
_NBUF = 2
_SB = 25
_NBB = 32
_K = 5
_SBP = _SB // _K
_BP = _B // _K


def _sc_gather(xs, table, part):
    b_per_w = _BP // _NW
    nchunks = b_per_w // _CHUNK

    mesh = plsc.VectorSubcoreMesh(core_axis_name="c", subcore_axis_name="s")

    @functools.partial(
        pl.kernel,
        out_type=jax.ShapeDtypeStruct((_BP, _D), jnp.float32),
        mesh=mesh,
        scratch_types=[
            pltpu.VMEM((b_per_w,), jnp.int32),
            pltpu.VMEM((b_per_w,), jnp.int32),
            pltpu.VMEM((_NBUF, _CHUNK, _D), jnp.float32),
            pltpu.SemaphoreType.DMA,
            pltpu.SemaphoreType.DMA,
            pltpu.SemaphoreType.DMA,
            pltpu.SemaphoreType.DMA,
        ],
        compiler_params=pltpu.CompilerParams(
            use_tc_tiling_on_sc=False, needs_layout_passes=False
        ),
    )
    def emb(idx_hbm, table_hbm, out_hbm, idx_r, idx_v, rows_v, g0, g1, s0, s1):
        wid = lax.axis_index("s") * _NC + lax.axis_index("c")
        base = wid * b_per_w
        pltpu.sync_copy(
            idx_hbm.at[pl.ds(part * _BP + base, b_per_w)], idx_r
        )

        def remap(v):
            return (v & -2048) + ((v & 1023) << 1) + ((v >> 10) & 1)

        ii2 = lax.iota(jnp.int32, 16) * 2

        @plsc.parallel_loop(0, b_per_w // 256)
        def _(blk):
            for g in range(8):
                src = blk * 256 + g * 16
                a = remap(idx_r[pl.ds(src, 16)])
                b = remap(idx_r[pl.ds(src + 128, 16)])
                pos = ii2 + (blk * 256 + g * 32)
                plsc.store_scatter(idx_v, [pos], a)
                plsc.store_scatter(idx_v, [pos + 1], b)

        gsem = (g0, g1)
        ssem = (s0, s1)

        def start_gather(slot, c):
            pltpu.make_async_copy(
                table_hbm.at[idx_v.at[pl.ds(c * _CHUNK, _CHUNK)]],
                rows_v.at[slot],
                gsem[slot],
            ).start()

        def wait_gather(slot):
            pltpu.make_async_copy(
                table_hbm.at[idx_v.at[pl.ds(0, _CHUNK)]],
                rows_v.at[slot],
                gsem[slot],
            ).wait()

        def start_scatter(slot, c):
            pltpu.make_async_copy(
                rows_v.at[slot],
                out_hbm.at[pl.ds(base + c * _CHUNK, _CHUNK)],
                ssem[slot],
            ).start()

        def wait_scatter(slot):
            pltpu.make_async_copy(
                rows_v.at[slot],
                out_hbm.at[pl.ds(base, _CHUNK)],
                ssem[slot],
            ).wait()

        start_gather(0, 0)
        start_gather(1, 1)

        @pl.loop(0, (nchunks - 2) // 2)
        def _(i):
            c0 = i * 2
            for b in range(_NBUF):
                wait_gather(b)
                start_scatter(b, c0 + b)
            for b in range(_NBUF):
                wait_scatter(b)
                start_gather(b, c0 + 2 + b)

        for b in range(_NBUF):
            wait_gather(b)
            start_scatter(b, nchunks - 2 + b)
        for b in range(_NBUF):
            wait_scatter(b)

    return emb(xs, table)


def _tc_relayout_body(tref, oref):
    blk = tref[...] * _SCALE
    for h in range(4):
        sub = blk[:, h * 2048 : (h + 1) * 2048]
        lo = jnp.transpose(sub[:, :1024], (1, 0))
        hi = jnp.transpose(sub[:, 1024:], (1, 0))
        oref[h * 128 : (h + 1) * 128] = jnp.concatenate(
            [lo, hi], axis=1
        ).reshape(128, 8, 128)


def _tc_relayout(table_t):
    return pl.pallas_call(
        _tc_relayout_body,
        grid=(123,),
        in_specs=[pl.BlockSpec((_D, 8192), lambda j: (0, j))],
        out_specs=pl.BlockSpec((512, 8, 128), lambda j: (j, 0, 0)),
        out_shape=jax.ShapeDtypeStruct((62976, 8, 128), jnp.float32),
        compiler_params=pltpu.CompilerParams(
            dimension_semantics=("arbitrary",)
        ),
    )(table_t)


def _tc_unpack(rows_ref, out_ref):
    for tt in range(8):
        pt = jnp.transpose(rows_ref[tt], (1, 0))
        for j in range(2):
            out_ref[2 * (tt % 4) + j, :, tt // 4, :, :] = pt[
                j * _D : (j + 1) * _D, :
            ].reshape(8, 8, 128)


def _tc_body_first(rows_ref, out_ref):
    _tc_unpack(rows_ref, out_ref)


def _tc_body(prev_ref, rows_ref, out_ref):
    del prev_ref
    _tc_unpack(rows_ref, out_ref)


def _tc_transpose(rows3, part, prev):
    out_shape = jax.ShapeDtypeStruct((200, 8, _NBB, 8, 128), jnp.float32)
    grid = (_SBP, _NBB // 2)

    def in_map(sb, bc):
        return (sb * (_NBB // 2) + bc, 0, 0)

    def out_map2(sb, bc, _part=part):
        return (_part * _SBP + sb, 0, bc, 0, 0)

    rows_spec = pl.BlockSpec((8, 128, 128), in_map)
    out_spec = pl.BlockSpec((8, 8, 2, 8, 128), out_map2)
    if prev is None:
        return pl.pallas_call(
            _tc_body_first,
            grid=grid,
            in_specs=[rows_spec],
            out_specs=out_spec,
            out_shape=out_shape,
            compiler_params=pltpu.CompilerParams(
                dimension_semantics=("arbitrary", "arbitrary")
            ),
        )(rows3)
    return pl.pallas_call(
        _tc_body,
        grid=grid,
        in_specs=[pl.BlockSpec(memory_space=pl.ANY), rows_spec],
        out_specs=out_spec,
        out_shape=out_shape,
        input_output_aliases={0: 0},
        compiler_params=pltpu.CompilerParams(
            dimension_semantics=("arbitrary", "arbitrary")
        ),
    )(prev, rows3)


@jax.jit
def _lookup(xs, table):
    table = _tc_relayout(table.T).reshape(1007616, _D)
    rows = [_sc_gather(xs, table, p) for p in range(_K)]
    out = None
    for p in range(_K):
        out = _tc_transpose(rows[p].reshape(_BP // 256, 128, 128), p, out)
    return out


def kernel(x, table):
    xs = x.astype(jnp.int32).reshape(_NBB, 128, _SB, 8).transpose(2, 0, 3, 1)
    out5 = _lookup(xs.reshape(-1), table)
    return out5.transpose(2, 4, 0, 1, 3).reshape(4096, 200, _D)


# speedup vs baseline: 1.8717x; 1.0497x over previous
import functools

import jax
import jax.numpy as jnp
from jax import lax
from jax.experimental import pallas as pl
from jax.experimental.pallas import tpu as pltpu
from jax.experimental.pallas import tpu_sc as plsc

_D = 64
_SCALE = 8.0
_NC, _NS = 2, 16
_NW = _NC * _NS
_B = 819200
_CHUNK = 512
_NBUF = 2
_SB = 25
_NBB = 32
_K = 5
_SBP = _SB // _K
_BP = _B // _K


def _sc_gather(xs, table, part):
    b_per_w = _BP // _NW
    nchunks = b_per_w // _CHUNK

    mesh = plsc.VectorSubcoreMesh(core_axis_name="c", subcore_axis_name="s")

    @functools.partial(
        pl.kernel,
        out_type=jax.ShapeDtypeStruct((_BP, _D), jnp.float32),
        mesh=mesh,
        scratch_types=[
            pltpu.VMEM((b_per_w,), jnp.int32),
            pltpu.VMEM((b_per_w,), jnp.int32),
            pltpu.VMEM((_NBUF, _CHUNK, _D), jnp.float32),
            pltpu.SemaphoreType.DMA,
            pltpu.SemaphoreType.DMA,
            pltpu.SemaphoreType.DMA,
            pltpu.SemaphoreType.DMA,
        ],
        compiler_params=pltpu.CompilerParams(
            use_tc_tiling_on_sc=False, needs_layout_passes=False
        ),
    )
    def emb(idx_hbm, table_hbm, out_hbm, idx_r, idx_v, rows_v, g0, g1, s0, s1):
        wid = lax.axis_index("s") * _NC + lax.axis_index("c")
        base = wid * b_per_w
        pltpu.sync_copy(
            idx_hbm.at[pl.ds(part * _BP + base, b_per_w)], idx_r
        )

        def remap(v):
            return (v & -2048) + ((v & 1023) << 1) + ((v >> 10) & 1)

        ii2 = lax.iota(jnp.int32, 16) * 2

        @plsc.parallel_loop(0, b_per_w // 256)
        def _(blk):
            for g in range(8):
                src = blk * 256 + g * 16
                a = remap(idx_r[pl.ds(src, 16)])
                b = remap(idx_r[pl.ds(src + 128, 16)])
                pos = ii2 + (blk * 256 + g * 32)
                plsc.store_scatter(idx_v, [pos], a)
                plsc.store_scatter(idx_v, [pos + 1], b)

        gsem = (g0, g1)
        ssem = (s0, s1)

        def start_gather(slot, c):
            pltpu.make_async_copy(
                table_hbm.at[idx_v.at[pl.ds(c * _CHUNK, _CHUNK)]],
                rows_v.at[slot],
                gsem[slot],
            ).start()

        def wait_gather(slot):
            pltpu.make_async_copy(
                table_hbm.at[idx_v.at[pl.ds(0, _CHUNK)]],
                rows_v.at[slot],
                gsem[slot],
            ).wait()

        def start_scatter(slot, c):
            pltpu.make_async_copy(
                rows_v.at[slot],
                out_hbm.at[pl.ds(base + c * _CHUNK, _CHUNK)],
                ssem[slot],
            ).start()

        def wait_scatter(slot):
            pltpu.make_async_copy(
                rows_v.at[slot],
                out_hbm.at[pl.ds(base, _CHUNK)],
                ssem[slot],
            ).wait()

        start_gather(0, 0)
        start_gather(1, 1)

        @pl.loop(0, (nchunks - 2) // 2)
        def _(i):
            c0 = i * 2
            for b in range(_NBUF):
                wait_gather(b)
                start_scatter(b, c0 + b)
            for b in range(_NBUF):
                wait_scatter(b)
                start_gather(b, c0 + 2 + b)

        for b in range(_NBUF):
            wait_gather(b)
            start_scatter(b, nchunks - 2 + b)
        for b in range(_NBUF):
            wait_scatter(b)

    return emb(xs, table)


def _tc_relayout_body(tref, oref):
    blk = tref[...] * _SCALE
    for h in range(8):
        sub = blk[:, h * 2048 : (h + 1) * 2048]
        lo = jnp.transpose(sub[:, :1024], (1, 0))
        hi = jnp.transpose(sub[:, 1024:], (1, 0))
        oref[h * 128 : (h + 1) * 128] = jnp.concatenate(
            [lo, hi], axis=1
        ).reshape(128, 8, 128)


def _tc_relayout(table_t):
    return pl.pallas_call(
        _tc_relayout_body,
        grid=(62,),
        in_specs=[pl.BlockSpec((_D, 16384), lambda j: (0, j))],
        out_specs=pl.BlockSpec((1024, 8, 128), lambda j: (j, 0, 0)),
        out_shape=jax.ShapeDtypeStruct((63488, 8, 128), jnp.float32),
        compiler_params=pltpu.CompilerParams(
            dimension_semantics=("arbitrary",)
        ),
    )(table_t)


def _tc_unpack(rows_ref, out_ref):
    for tt in range(8):
        pt = jnp.transpose(rows_ref[tt], (1, 0))
        for j in range(2):
            out_ref[2 * (tt % 4) + j, :, tt // 4, :, :] = pt[
                j * _D : (j + 1) * _D, :
            ].reshape(8, 8, 128)


def _tc_body_first(rows_ref, out_ref):
    _tc_unpack(rows_ref, out_ref)


def _tc_body(prev_ref, rows_ref, out_ref):
    del prev_ref
    _tc_unpack(rows_ref, out_ref)


def _tc_transpose(rows3, part, prev):
    out_shape = jax.ShapeDtypeStruct((200, 8, _NBB, 8, 128), jnp.float32)
    grid = (_SBP, _NBB // 2)

    def in_map(sb, bc):
        return (sb * (_NBB // 2) + bc, 0, 0)

    def out_map2(sb, bc, _part=part):
        return (_part * _SBP + sb, 0, bc, 0, 0)

    rows_spec = pl.BlockSpec((8, 128, 128), in_map)
    out_spec = pl.BlockSpec((8, 8, 2, 8, 128), out_map2)
    if prev is None:
        return pl.pallas_call(
            _tc_body_first,
            grid=grid,
            in_specs=[rows_spec],
            out_specs=out_spec,
            out_shape=out_shape,
            compiler_params=pltpu.CompilerParams(
                dimension_semantics=("arbitrary", "arbitrary")
            ),
        )(rows3)
    return pl.pallas_call(
        _tc_body,
        grid=grid,
        in_specs=[pl.BlockSpec(memory_space=pl.ANY), rows_spec],
        out_specs=out_spec,
        out_shape=out_shape,
        input_output_aliases={0: 0},
        compiler_params=pltpu.CompilerParams(
            dimension_semantics=("arbitrary", "arbitrary")
        ),
    )(prev, rows3)


@jax.jit
def _lookup(xs, table):
    table = _tc_relayout(table.T).reshape(1015808, _D)
    rows = [_sc_gather(xs, table, p) for p in range(_K)]
    out = None
    for p in range(_K):
        out = _tc_transpose(rows[p].reshape(_BP // 256, 128, 128), p, out)
    return out


def kernel(x, table):
    xs = x.astype(jnp.int32).reshape(_NBB, 128, _SB, 8).transpose(2, 0, 3, 1)
    out5 = _lookup(xs.reshape(-1), table)
    return out5.transpose(2, 4, 0, 1, 3).reshape(4096, 200, _D)


# speedup vs baseline: 1.9081x; 1.0194x over previous
import functools

import jax
import jax.numpy as jnp
from jax import lax
from jax.experimental import pallas as pl
from jax.experimental.pallas import tpu as pltpu
from jax.experimental.pallas import tpu_sc as plsc

_D = 64
_SCALE = 8.0
_NC, _NS = 2, 16
_NW = _NC * _NS
_B = 819200
_CHUNK = 512
_NBUF = 2
_SB = 25
_NBB = 32
_K = 5
_SBP = _SB // _K
_BP = _B // _K


def _sc_gather(xs, table, part):
    b_per_w = _BP // _NW
    nchunks = b_per_w // _CHUNK

    mesh = plsc.VectorSubcoreMesh(core_axis_name="c", subcore_axis_name="s")

    @functools.partial(
        pl.kernel,
        out_type=jax.ShapeDtypeStruct((_BP, _D), jnp.float32),
        mesh=mesh,
        scratch_types=[
            pltpu.VMEM((b_per_w,), jnp.int32),
            pltpu.VMEM((b_per_w,), jnp.int32),
            pltpu.VMEM((_NBUF, _CHUNK, _D), jnp.float32),
            pltpu.SemaphoreType.DMA,
            pltpu.SemaphoreType.DMA,
            pltpu.SemaphoreType.DMA,
            pltpu.SemaphoreType.DMA,
        ],
        compiler_params=pltpu.CompilerParams(
            use_tc_tiling_on_sc=False, needs_layout_passes=False
        ),
    )
    def emb(idx_hbm, table_hbm, out_hbm, idx_r, idx_v, rows_v, g0, g1, s0, s1):
        wid = lax.axis_index("s") * _NC + lax.axis_index("c")
        base = wid * b_per_w
        pltpu.sync_copy(
            idx_hbm.at[pl.ds(part * _BP + base, b_per_w)], idx_r
        )

        def remap(v):
            return (v & -2048) + ((v & 1023) << 1) + ((v >> 10) & 1)

        ii2 = lax.iota(jnp.int32, 16) * 2

        @plsc.parallel_loop(0, b_per_w // 256)
        def _(blk):
            for g in range(8):
                src = blk * 256 + g * 16
                a = remap(idx_r[pl.ds(src, 16)])
                b = remap(idx_r[pl.ds(src + 128, 16)])
                pos = ii2 + (blk * 256 + g * 32)
                plsc.store_scatter(idx_v, [pos], a)
                plsc.store_scatter(idx_v, [pos + 1], b)

        gsem = (g0, g1)
        ssem = (s0, s1)

        def start_gather(slot, c):
            pltpu.make_async_copy(
                table_hbm.at[idx_v.at[pl.ds(c * _CHUNK, _CHUNK)]],
                rows_v.at[slot],
                gsem[slot],
            ).start()

        def wait_gather(slot):
            pltpu.make_async_copy(
                table_hbm.at[idx_v.at[pl.ds(0, _CHUNK)]],
                rows_v.at[slot],
                gsem[slot],
            ).wait()

        def start_scatter(slot, c):
            pltpu.make_async_copy(
                rows_v.at[slot],
                out_hbm.at[pl.ds(base + c * _CHUNK, _CHUNK)],
                ssem[slot],
            ).start()

        def wait_scatter(slot):
            pltpu.make_async_copy(
                rows_v.at[slot],
                out_hbm.at[pl.ds(base, _CHUNK)],
                ssem[slot],
            ).wait()

        start_gather(0, 0)
        start_gather(1, 1)

        @pl.loop(0, (nchunks - 2) // 2)
        def _(i):
            c0 = i * 2
            for b in range(_NBUF):
                wait_gather(b)
                start_scatter(b, c0 + b)
            for b in range(_NBUF):
                wait_scatter(b)
                start_gather(b, c0 + 2 + b)

        for b in range(_NBUF):
            wait_gather(b)
            start_scatter(b, nchunks - 2 + b)
        for b in range(_NBUF):
            wait_scatter(b)

    return emb(xs, table)


def _tc_relayout_body(tref, oref):
    blk = tref[...] * _SCALE
    for h in range(16):
        sub = blk[:, h * 2048 : (h + 1) * 2048]
        lo = jnp.transpose(sub[:, :1024], (1, 0))
        hi = jnp.transpose(sub[:, 1024:], (1, 0))
        oref[h * 128 : (h + 1) * 128] = jnp.concatenate(
            [lo, hi], axis=1
        ).reshape(128, 8, 128)


def _tc_relayout(table_t):
    return pl.pallas_call(
        _tc_relayout_body,
        grid=(31,),
        in_specs=[pl.BlockSpec((_D, 32768), lambda j: (0, j))],
        out_specs=pl.BlockSpec((2048, 8, 128), lambda j: (j, 0, 0)),
        out_shape=jax.ShapeDtypeStruct((63488, 8, 128), jnp.float32),
        compiler_params=pltpu.CompilerParams(
            dimension_semantics=("arbitrary",)
        ),
    )(table_t)


def _tc_unpack(rows_ref, out_ref):
    for tt in range(8):
        pt = jnp.transpose(rows_ref[tt], (1, 0))
        for j in range(2):
            out_ref[2 * (tt % 4) + j, :, tt // 4, :, :] = pt[
                j * _D : (j + 1) * _D, :
            ].reshape(8, 8, 128)


def _tc_body_first(rows_ref, out_ref):
    _tc_unpack(rows_ref, out_ref)


def _tc_body(prev_ref, rows_ref, out_ref):
    del prev_ref
    _tc_unpack(rows_ref, out_ref)


def _tc_transpose(rows3, part, prev):
    out_shape = jax.ShapeDtypeStruct((200, 8, _NBB, 8, 128), jnp.float32)
    grid = (_SBP, _NBB // 2)

    def in_map(sb, bc):
        return (sb * (_NBB // 2) + bc, 0, 0)

    def out_map2(sb, bc, _part=part):
        return (_part * _SBP + sb, 0, bc, 0, 0)

    rows_spec = pl.BlockSpec((8, 128, 128), in_map)
    out_spec = pl.BlockSpec((8, 8, 2, 8, 128), out_map2)
    if prev is None:
        return pl.pallas_call(
            _tc_body_first,
            grid=grid,
            in_specs=[rows_spec],
            out_specs=out_spec,
            out_shape=out_shape,
            compiler_params=pltpu.CompilerParams(
                dimension_semantics=("arbitrary", "arbitrary")
            ),
        )(rows3)
    return pl.pallas_call(
        _tc_body,
        grid=grid,
        in_specs=[pl.BlockSpec(memory_space=pl.ANY), rows_spec],
        out_specs=out_spec,
        out_shape=out_shape,
        input_output_aliases={0: 0},
        compiler_params=pltpu.CompilerParams(
            dimension_semantics=("arbitrary", "arbitrary")
        ),
    )(prev, rows3)


@jax.jit
def _lookup(xs, table):
    table = _tc_relayout(table.T).reshape(1015808, _D)
    rows = [_sc_gather(xs, table, p) for p in range(_K)]
    out = None
    for p in range(_K):
        out = _tc_transpose(rows[p].reshape(_BP // 256, 128, 128), p, out)
    return out


def kernel(x, table):
    xs = x.astype(jnp.int32).reshape(_NBB, 128, _SB, 8).transpose(2, 0, 3, 1)
    out5 = _lookup(xs.reshape(-1), table)
    return out5.transpose(2, 4, 0, 1, 3).reshape(4096, 200, _D)
